# speedup vs baseline: 1.5500x; 1.5500x over previous
import jax
import jax.numpy as jnp
from jax.experimental import pallas as pl
from jax.experimental.pallas import tpu as pltpu

_B = 64
_L = 128
_DM = 768
_DF = 1536
_DL = 2048
_E = 8
_NP = _B * 2
_G = _NP + _B


def _gate_kernel(dkp_ref, cn_ref, w1_ref, b1_ref, w2_ref, b2_ref, w3_ref,
                 b3_ref, sat_ref, eat_ref, wat_ref, pp_ref):
    dkp = dkp_ref[...]
    h1 = jnp.maximum(
        jnp.dot(dkp, w1_ref[...], preferred_element_type=jnp.float32)
        + b1_ref[...], 0.0)
    h2 = jnp.maximum(cn_ref[...] * w2_ref[...] + b2_ref[...], 0.0)
    h = h1 + h2
    logits = (jnp.dot(h, w3_ref[...], preferred_element_type=jnp.float32)
              + b3_ref[...])

    lane8 = jax.lax.broadcasted_iota(jnp.int32, (_B, _E), 1)
    m1 = jnp.max(logits, axis=1, keepdims=True)
    i1 = jnp.min(jnp.where(logits == m1, lane8, _E), axis=1, keepdims=True)
    oh1 = lane8 == i1
    masked = jnp.where(oh1, -jnp.inf, logits)
    m2 = jnp.max(masked, axis=1, keepdims=True)
    i2 = jnp.min(jnp.where(masked == m2, lane8, _E), axis=1, keepdims=True)
    oh2 = lane8 == i2
    sel = oh1 | oh2

    p = jnp.exp(logits - m1)
    p = p / jnp.sum(p, axis=1, keepdims=True)
    pm = jnp.where(sel, p, 0.0)
    gates = pm / (jnp.sum(pm, axis=1, keepdims=True) + 1e-9)

    mf = sel.astype(jnp.float32)
    r0 = jax.lax.broadcasted_iota(jnp.int32, (_B, _B), 0)
    c0 = jax.lax.broadcasted_iota(jnp.int32, (_B, _B), 1)
    ltri = (c0 < r0).astype(jnp.float32)
    rank = jnp.dot(ltri, mf, preferred_element_type=jnp.float32)
    count = jnp.sum(mf, axis=0, keepdims=True)
    re = jax.lax.broadcasted_iota(jnp.int32, (_E, _E), 0)
    ce = jax.lax.broadcasted_iota(jnp.int32, (_E, _E), 1)
    utri = (re < ce).astype(jnp.float32)
    off = jnp.dot(count, utri, preferred_element_type=jnp.float32)
    pos = (off + rank).astype(jnp.int32)
    pos = jnp.where(sel, pos, _G + 1)

    lane_g = jax.lax.broadcasted_iota(jnp.int32, (_B, _E, _G), 2)
    hit = pos[:, :, None] == lane_g
    s3 = jax.lax.broadcasted_iota(jnp.int32, (_B, _E, _G), 0)
    e3 = jax.lax.broadcasted_iota(jnp.int32, (_B, _E, _G), 1)
    sat_s = jnp.sum(jnp.sum(jnp.where(hit, s3, 0), axis=0), axis=0)
    eat_s = jnp.sum(jnp.sum(jnp.where(hit, e3, 0), axis=0), axis=0)
    wat_s = jnp.sum(jnp.sum(jnp.where(hit, gates[:, :, None], 0.0), axis=0),
                    axis=0)

    lane1 = jax.lax.broadcasted_iota(jnp.int32, (1, _G), 1)
    is_gen = lane1 >= _NP
    sat_ref[...] = jnp.where(is_gen, lane1 - _NP, sat_s[None, :])
    eat_ref[...] = jnp.where(is_gen, _E, eat_s[None, :])
    wat_ref[...] = jnp.where(is_gen, 1.0, wat_s[None, :])

    p0 = jnp.sum(jnp.where(oh1, pos, 0), axis=1, keepdims=True)
    p1 = jnp.sum(jnp.where(oh2, pos, 0), axis=1, keepdims=True)
    pp_ref[...] = jnp.where(lane8 == 0, p0, jnp.where(lane8 == 1, p1, 0))


def _mlp_kernel(sat_ref, eat_ref, x_ref, wi_ref, bi_ref, wo_ref, bo_ref,
                lg_ref, lb_ref, wat_ref, out_ref):
    g = pl.program_id(0)
    x = x_ref[0]
    h = jnp.maximum(
        jnp.dot(x, wi_ref[0], preferred_element_type=jnp.float32)
        + bi_ref[0], 0.0)
    o = (jnp.dot(h, wo_ref[0], preferred_element_type=jnp.float32)
         + bo_ref[0] + x)
    mu = jnp.mean(o, axis=1, keepdims=True)
    var = jnp.mean((o - mu) ** 2, axis=1, keepdims=True)
    o = (o - mu) / jnp.sqrt(var + 1e-5) * lg_ref[0] + lb_ref[0]
    lane = jax.lax.broadcasted_iota(jnp.int32, (1, _G), 1)
    w = jnp.sum(jnp.where(lane == g, wat_ref[...], 0.0))
    out_ref[0] = o * w


def _combine_kernel(p0_ref, p1_ref, a_ref, b_ref, gen_ref, out_ref):
    tot = a_ref[0] + b_ref[0]
    tot = tot.astype(jnp.bfloat16).astype(jnp.float32)
    out_ref[0] = gen_ref[0] + tot


def kernel(cycle_curve_data, cycle_numbers, DKP_embeddings, gate_W1, gate_b1,
           gate_W2, gate_b2, gate_W3, gate_b3, exp_Wi, exp_bi, exp_Wo, exp_bo,
           exp_g, exp_b, gen_Wi, gen_bi, gen_Wo, gen_bo, gen_g, gen_b):
    sat, eat, wat, pp = pl.pallas_call(
        _gate_kernel,
        out_shape=(
            jax.ShapeDtypeStruct((1, _G), jnp.int32),
            jax.ShapeDtypeStruct((1, _G), jnp.int32),
            jax.ShapeDtypeStruct((1, _G), jnp.float32),
            jax.ShapeDtypeStruct((_B, _E), jnp.int32),
        ),
    )(DKP_embeddings, cycle_numbers, gate_W1, gate_b1.reshape(1, _DF),
      gate_W2, gate_b2.reshape(1, _DF), gate_W3, gate_b3.reshape(1, _E))

    sat1 = sat.reshape(_G)
    eat1 = eat.reshape(_G)
    p0 = pp[:, 0]
    p1 = pp[:, 1]

    wi_all = jnp.concatenate([exp_Wi, gen_Wi[None]], axis=0)
    bi_all = jnp.concatenate([exp_bi, gen_bi[None]], axis=0)[:, None, :]
    wo_all = jnp.concatenate([exp_Wo, gen_Wo[None]], axis=0)
    bo_all = jnp.concatenate([exp_bo, gen_bo[None]], axis=0)[:, None, :]
    lg_all = jnp.concatenate([exp_g, gen_g[None]], axis=0)[:, None, :]
    lb_all = jnp.concatenate([exp_b, gen_b[None]], axis=0)[:, None, :]

    slots = pl.pallas_call(
        _mlp_kernel,
        grid_spec=pltpu.PrefetchScalarGridSpec(
            num_scalar_prefetch=2,
            grid=(_G,),
            in_specs=[
                pl.BlockSpec((1, _L, _DM), lambda g, sat, eat: (sat[g], 0, 0)),
                pl.BlockSpec((1, _DM, _DF), lambda g, sat, eat: (eat[g], 0, 0)),
                pl.BlockSpec((1, 1, _DF), lambda g, sat, eat: (eat[g], 0, 0)),
                pl.BlockSpec((1, _DF, _DM), lambda g, sat, eat: (eat[g], 0, 0)),
                pl.BlockSpec((1, 1, _DM), lambda g, sat, eat: (eat[g], 0, 0)),
                pl.BlockSpec((1, 1, _DM), lambda g, sat, eat: (eat[g], 0, 0)),
                pl.BlockSpec((1, 1, _DM), lambda g, sat, eat: (eat[g], 0, 0)),
                pl.BlockSpec((1, _G), lambda g, sat, eat: (0, 0)),
            ],
            out_specs=pl.BlockSpec((1, _L, _DM), lambda g, sat, eat: (g, 0, 0)),
        ),
        out_shape=jax.ShapeDtypeStruct((_G, _L, _DM), jnp.float32),
    )(sat1, eat1, cycle_curve_data, wi_all, bi_all, wo_all, bo_all,
      lg_all, lb_all, wat)

    final = pl.pallas_call(
        _combine_kernel,
        grid_spec=pltpu.PrefetchScalarGridSpec(
            num_scalar_prefetch=2,
            grid=(_B,),
            in_specs=[
                pl.BlockSpec((1, _L, _DM), lambda s, p0, p1: (p0[s], 0, 0)),
                pl.BlockSpec((1, _L, _DM), lambda s, p0, p1: (p1[s], 0, 0)),
                pl.BlockSpec((1, _L, _DM), lambda s, p0, p1: (_NP + s, 0, 0)),
            ],
            out_specs=pl.BlockSpec((1, _L, _DM), lambda s, p0, p1: (s, 0, 0)),
        ),
        out_shape=jax.ShapeDtypeStruct((_B, _L, _DM), jnp.float32),
    )(p0, p1, slots, slots, slots)

    return (final, jnp.float32(0.0))


# speedup vs baseline: 2.5770x; 1.6626x over previous
import jax
import jax.numpy as jnp
from jax.experimental import pallas as pl
from jax.experimental.pallas import tpu as pltpu

_B = 64
_L = 128
_DM = 768
_DF = 1536
_DL = 2048
_E = 8


def _gate_kernel(dkp_ref, cn_ref, w1_ref, b1_ref, w2_ref, b2_ref, w3_ref,
                 b3_ref, ee_ref, gates_ref):
    dkp = dkp_ref[...]
    h1 = jnp.maximum(
        jnp.dot(dkp, w1_ref[...], preferred_element_type=jnp.float32)
        + b1_ref[...], 0.0)
    h2 = jnp.maximum(cn_ref[...] * w2_ref[...] + b2_ref[...], 0.0)
    h = h1 + h2
    logits = (jnp.dot(h, w3_ref[...], preferred_element_type=jnp.float32)
              + b3_ref[...])

    lane8 = jax.lax.broadcasted_iota(jnp.int32, (_B, _E), 1)
    m1 = jnp.max(logits, axis=1, keepdims=True)
    i1 = jnp.min(jnp.where(logits == m1, lane8, _E), axis=1, keepdims=True)
    oh1 = lane8 == i1
    masked = jnp.where(oh1, -jnp.inf, logits)
    m2 = jnp.max(masked, axis=1, keepdims=True)
    i2 = jnp.min(jnp.where(masked == m2, lane8, _E), axis=1, keepdims=True)
    oh2 = lane8 == i2
    sel = oh1 | oh2

    p = jnp.exp(logits - m1)
    p = p / jnp.sum(p, axis=1, keepdims=True)
    pm = jnp.where(sel, p, 0.0)
    gates_ref[...] = pm / (jnp.sum(pm, axis=1, keepdims=True) + 1e-9)

    ee_ref[...] = jnp.where(lane8 == 0, i1, jnp.where(lane8 == 1, i2, 0))


def _moe_kernel(e0_ref, e1_ref, x_ref, gates_ref, wi_ref, bi_ref, wo_ref,
                bo_ref, lg_ref, lb_ref, out_ref):
    s = pl.program_id(0)
    x = x_ref[0]
    xb = x.astype(jnp.bfloat16)

    def mlp(e):
        h = jnp.maximum(
            jnp.dot(xb, wi_ref[e], preferred_element_type=jnp.float32)
            + bi_ref[e], 0.0)
        o = (jnp.dot(h.astype(jnp.bfloat16), wo_ref[e],
                     preferred_element_type=jnp.float32)
             + bo_ref[e] + x)
        mu = jnp.mean(o, axis=1, keepdims=True)
        var = jnp.mean((o - mu) ** 2, axis=1, keepdims=True)
        return (o - mu) / jnp.sqrt(var + 1e-5) * lg_ref[e] + lb_ref[e]

    e0 = e0_ref[s]
    e1 = e1_ref[s]
    lane8 = jax.lax.broadcasted_iota(jnp.int32, (1, _E), 1)
    grow = gates_ref[pl.ds(s, 1), :]
    w0 = jnp.sum(jnp.where(lane8 == e0, grow, 0.0))
    w1 = jnp.sum(jnp.where(lane8 == e1, grow, 0.0))
    tot = mlp(e0) * w0 + mlp(e1) * w1
    tot = tot.astype(jnp.bfloat16).astype(jnp.float32)
    out_ref[0] = mlp(_E) + tot


def kernel(cycle_curve_data, cycle_numbers, DKP_embeddings, gate_W1, gate_b1,
           gate_W2, gate_b2, gate_W3, gate_b3, exp_Wi, exp_bi, exp_Wo, exp_bo,
           exp_g, exp_b, gen_Wi, gen_bi, gen_Wo, gen_bo, gen_g, gen_b):
    ee, gates = pl.pallas_call(
        _gate_kernel,
        out_shape=(
            jax.ShapeDtypeStruct((_B, _E), jnp.int32),
            jax.ShapeDtypeStruct((_B, _E), jnp.float32),
        ),
    )(DKP_embeddings, cycle_numbers, gate_W1, gate_b1.reshape(1, _DF),
      gate_W2, gate_b2.reshape(1, _DF), gate_W3, gate_b3.reshape(1, _E))

    e0 = ee[:, 0]
    e1 = ee[:, 1]

    wi_all = jnp.concatenate(
        [exp_Wi, gen_Wi[None]], axis=0).astype(jnp.bfloat16)
    wo_all = jnp.concatenate(
        [exp_Wo, gen_Wo[None]], axis=0).astype(jnp.bfloat16)
    bi_all = jnp.concatenate([exp_bi, gen_bi[None]], axis=0)[:, None, :]
    bo_all = jnp.concatenate([exp_bo, gen_bo[None]], axis=0)[:, None, :]
    lg_all = jnp.concatenate([exp_g, gen_g[None]], axis=0)[:, None, :]
    lb_all = jnp.concatenate([exp_b, gen_b[None]], axis=0)[:, None, :]

    final = pl.pallas_call(
        _moe_kernel,
        grid_spec=pltpu.PrefetchScalarGridSpec(
            num_scalar_prefetch=2,
            grid=(_B,),
            in_specs=[
                pl.BlockSpec((1, _L, _DM), lambda s, e0, e1: (s, 0, 0)),
                pl.BlockSpec((_B, _E), lambda s, e0, e1: (0, 0)),
                pl.BlockSpec((_E + 1, _DM, _DF), lambda s, e0, e1: (0, 0, 0)),
                pl.BlockSpec((_E + 1, 1, _DF), lambda s, e0, e1: (0, 0, 0)),
                pl.BlockSpec((_E + 1, _DF, _DM), lambda s, e0, e1: (0, 0, 0)),
                pl.BlockSpec((_E + 1, 1, _DM), lambda s, e0, e1: (0, 0, 0)),
                pl.BlockSpec((_E + 1, 1, _DM), lambda s, e0, e1: (0, 0, 0)),
                pl.BlockSpec((_E + 1, 1, _DM), lambda s, e0, e1: (0, 0, 0)),
            ],
            out_specs=pl.BlockSpec((1, _L, _DM), lambda s, e0, e1: (s, 0, 0)),
        ),
        out_shape=jax.ShapeDtypeStruct((_B, _L, _DM), jnp.float32),
    )(e0, e1, cycle_curve_data, gates, wi_all, bi_all, wo_all, bo_all,
      lg_all, lb_all)

    return (final, jnp.float32(0.0))


# speedup vs baseline: 2.9396x; 1.1407x over previous
import jax
import jax.numpy as jnp
from jax.experimental import pallas as pl
from jax.experimental.pallas import tpu as pltpu

_B = 64
_L = 128
_DM = 768
_DF = 1536
_DL = 2048
_E = 8


def _gate_kernel(dkp_ref, cn_ref, w1_ref, b1_ref, w2_ref, b2_ref, w3_ref,
                 b3_ref, ee_ref, gates_ref):
    dkp = dkp_ref[...]
    h1 = jnp.maximum(
        jnp.dot(dkp, w1_ref[...], preferred_element_type=jnp.float32)
        + b1_ref[...], 0.0)
    h2 = jnp.maximum(cn_ref[...] * w2_ref[...] + b2_ref[...], 0.0)
    h = h1 + h2
    logits = (jnp.dot(h, w3_ref[...], preferred_element_type=jnp.float32)
              + b3_ref[...])

    lane8 = jax.lax.broadcasted_iota(jnp.int32, (_B, _E), 1)
    m1 = jnp.max(logits, axis=1, keepdims=True)
    i1 = jnp.min(jnp.where(logits == m1, lane8, _E), axis=1, keepdims=True)
    oh1 = lane8 == i1
    masked = jnp.where(oh1, -jnp.inf, logits)
    m2 = jnp.max(masked, axis=1, keepdims=True)
    i2 = jnp.min(jnp.where(masked == m2, lane8, _E), axis=1, keepdims=True)
    oh2 = lane8 == i2
    sel = oh1 | oh2

    p = jnp.exp(logits - m1)
    p = p / jnp.sum(p, axis=1, keepdims=True)
    pm = jnp.where(sel, p, 0.0)
    gates_ref[...] = pm / (jnp.sum(pm, axis=1, keepdims=True) + 1e-9)

    ee_ref[...] = jnp.where(lane8 == 0, i1, jnp.where(lane8 == 1, i2, 0))


_HW = _DM // 2
_HO = _DF // 2


def _moe_kernel(e0_ref, e1_ref, x_ref, gates_ref, ewi_ref, ewo_ref, gwi_ref,
                gwo_ref, bi_ref, bo_ref, lg_ref, lb_ref, out_ref,
                wi_bf, wo_bf, stg_i, stg_o, sem_i, sem_o):
    s = pl.program_id(0)

    @pl.when(s == 0)
    def _load_weights():
        wi_srcs = ([(ewi_ref, e, h, e) for e in range(_E) for h in range(2)]
                   + [(gwi_ref, 0, h, _E) for h in range(2)])
        wo_srcs = ([(ewo_ref, e, h, e) for e in range(_E) for h in range(2)]
                   + [(gwo_ref, 0, h, _E) for h in range(2)])

        def wi_cp(k):
            src, se, h, _ = wi_srcs[k]
            return pltpu.make_async_copy(
                src.at[se, pl.ds(h * _HW, _HW), :], stg_i.at[k % 2],
                sem_i.at[k % 2])

        def wo_cp(k):
            src, se, h, _ = wo_srcs[k]
            return pltpu.make_async_copy(
                src.at[se, pl.ds(h * _HO, _HO), :], stg_o.at[k % 2],
                sem_o.at[k % 2])

        wi_cp(0).start()
        wo_cp(0).start()
        n = len(wi_srcs)
        for k in range(n):
            if k + 1 < n:
                wi_cp(k + 1).start()
                wo_cp(k + 1).start()
            wi_cp(k).wait()
            _, _, h, de = wi_srcs[k]
            wi_bf[de, pl.ds(h * _HW, _HW), :] = stg_i[k % 2].astype(
                jnp.bfloat16)
            wo_cp(k).wait()
            _, _, h2, de2 = wo_srcs[k]
            wo_bf[de2, pl.ds(h2 * _HO, _HO), :] = stg_o[k % 2].astype(
                jnp.bfloat16)

    x = x_ref[0]
    xb = x.astype(jnp.bfloat16)

    def mlp(e):
        h = jnp.maximum(
            jnp.dot(xb, wi_bf[e], preferred_element_type=jnp.float32)
            + bi_ref[e], 0.0)
        o = (jnp.dot(h.astype(jnp.bfloat16), wo_bf[e],
                     preferred_element_type=jnp.float32)
             + bo_ref[e] + x)
        mu = jnp.mean(o, axis=1, keepdims=True)
        var = jnp.mean((o - mu) ** 2, axis=1, keepdims=True)
        return (o - mu) / jnp.sqrt(var + 1e-5) * lg_ref[e] + lb_ref[e]

    e0 = e0_ref[s]
    e1 = e1_ref[s]
    lane8 = jax.lax.broadcasted_iota(jnp.int32, (1, _E), 1)
    grow = gates_ref[pl.ds(s, 1), :]
    w0 = jnp.sum(jnp.where(lane8 == e0, grow, 0.0))
    w1 = jnp.sum(jnp.where(lane8 == e1, grow, 0.0))
    tot = mlp(e0) * w0 + mlp(e1) * w1
    tot = tot.astype(jnp.bfloat16).astype(jnp.float32)
    out_ref[0] = mlp(_E) + tot


def kernel(cycle_curve_data, cycle_numbers, DKP_embeddings, gate_W1, gate_b1,
           gate_W2, gate_b2, gate_W3, gate_b3, exp_Wi, exp_bi, exp_Wo, exp_bo,
           exp_g, exp_b, gen_Wi, gen_bi, gen_Wo, gen_bo, gen_g, gen_b):
    ee, gates = pl.pallas_call(
        _gate_kernel,
        out_shape=(
            jax.ShapeDtypeStruct((_B, _E), jnp.int32),
            jax.ShapeDtypeStruct((_B, _E), jnp.float32),
        ),
    )(DKP_embeddings, cycle_numbers, gate_W1, gate_b1.reshape(1, _DF),
      gate_W2, gate_b2.reshape(1, _DF), gate_W3, gate_b3.reshape(1, _E))

    e0 = ee[:, 0]
    e1 = ee[:, 1]

    bi_all = jnp.concatenate([exp_bi, gen_bi[None]], axis=0)[:, None, :]
    bo_all = jnp.concatenate([exp_bo, gen_bo[None]], axis=0)[:, None, :]
    lg_all = jnp.concatenate([exp_g, gen_g[None]], axis=0)[:, None, :]
    lb_all = jnp.concatenate([exp_b, gen_b[None]], axis=0)[:, None, :]

    final = pl.pallas_call(
        _moe_kernel,
        grid_spec=pltpu.PrefetchScalarGridSpec(
            num_scalar_prefetch=2,
            grid=(_B,),
            in_specs=[
                pl.BlockSpec((1, _L, _DM), lambda s, e0, e1: (s, 0, 0)),
                pl.BlockSpec((_B, _E), lambda s, e0, e1: (0, 0)),
                pl.BlockSpec(memory_space=pl.ANY),
                pl.BlockSpec(memory_space=pl.ANY),
                pl.BlockSpec(memory_space=pl.ANY),
                pl.BlockSpec(memory_space=pl.ANY),
                pl.BlockSpec((_E + 1, 1, _DF), lambda s, e0, e1: (0, 0, 0)),
                pl.BlockSpec((_E + 1, 1, _DM), lambda s, e0, e1: (0, 0, 0)),
                pl.BlockSpec((_E + 1, 1, _DM), lambda s, e0, e1: (0, 0, 0)),
                pl.BlockSpec((_E + 1, 1, _DM), lambda s, e0, e1: (0, 0, 0)),
            ],
            out_specs=pl.BlockSpec((1, _L, _DM), lambda s, e0, e1: (s, 0, 0)),
            scratch_shapes=[
                pltpu.VMEM((_E + 1, _DM, _DF), jnp.bfloat16),
                pltpu.VMEM((_E + 1, _DF, _DM), jnp.bfloat16),
                pltpu.VMEM((2, _HW, _DF), jnp.float32),
                pltpu.VMEM((2, _HO, _DM), jnp.float32),
                pltpu.SemaphoreType.DMA((2,)),
                pltpu.SemaphoreType.DMA((2,)),
            ],
        ),
        out_shape=jax.ShapeDtypeStruct((_B, _L, _DM), jnp.float32),
    )(e0, e1, cycle_curve_data, gates, exp_Wi, exp_Wo,
      gen_Wi.reshape(1, _DM, _DF), gen_Wo.reshape(1, _DF, _DM),
      bi_all, bo_all, lg_all, lb_all)

    return (final, jnp.float32(0.0))


# speedup vs baseline: 2.9602x; 1.0070x over previous
import jax
import jax.numpy as jnp
from jax.experimental import pallas as pl
from jax.experimental.pallas import tpu as pltpu

_B = 64
_L = 128
_DM = 768
_DF = 1536
_DL = 2048
_E = 8

_HW = _DM // 2
_HO = _DF // 2
_C1 = 256


def _moe_kernel(x_ref, dkp_ref, cn_ref, w1_ref, b1_ref, w2_ref, b2_ref,
                w3_ref, b3_ref, ewi_ref, ewo_ref, gwi_ref, gwo_ref,
                bi_ref, bo_ref, lg_ref, lb_ref, out_ref,
                wi_bf, wo_bf, stg_i, stg_o, stg_w1, gates_scr, ee_scr,
                ee_smem, sem_i, sem_o, sem_w1, sem_ee):
    s = pl.program_id(0)

    @pl.when(s == 0)
    def _prologue():
        wi_srcs = ([(ewi_ref, e, h, e) for e in range(_E) for h in range(2)]
                   + [(gwi_ref, 0, h, _E) for h in range(2)])
        wo_srcs = ([(ewo_ref, e, h, e) for e in range(_E) for h in range(2)]
                   + [(gwo_ref, 0, h, _E) for h in range(2)])

        def wi_cp(k):
            src, se, h, _ = wi_srcs[k]
            return pltpu.make_async_copy(
                src.at[se, pl.ds(h * _HW, _HW), :], stg_i.at[k % 2],
                sem_i.at[k % 2])

        def wo_cp(k):
            src, se, h, _ = wo_srcs[k]
            return pltpu.make_async_copy(
                src.at[se, pl.ds(h * _HO, _HO), :], stg_o.at[k % 2],
                sem_o.at[k % 2])

        def w1_cp(c):
            return pltpu.make_async_copy(
                w1_ref.at[pl.ds(c * _C1, _C1), :], stg_w1.at[c % 2],
                sem_w1.at[c % 2])

        wi_cp(0).start()
        wo_cp(0).start()
        w1_cp(0).start()
        w1_cp(1).start()

        dkp = dkp_ref[...]
        nc = _DL // _C1
        h_acc = jnp.zeros((_B, _DF), jnp.float32)
        for c in range(nc):
            w1_cp(c).wait()
            chunk = stg_w1[c % 2]
            h_acc = h_acc + jnp.dot(
                dkp[:, c * _C1:(c + 1) * _C1], chunk,
                preferred_element_type=jnp.float32)
            if c + 2 < nc:
                w1_cp(c + 2).start()
        h1 = jnp.maximum(h_acc + b1_ref[...], 0.0)
        h2 = jnp.maximum(cn_ref[...] * w2_ref[...] + b2_ref[...], 0.0)
        h = h1 + h2
        logits = (jnp.dot(h, w3_ref[...], preferred_element_type=jnp.float32)
                  + b3_ref[...])

        lane8 = jax.lax.broadcasted_iota(jnp.int32, (_B, _E), 1)
        m1 = jnp.max(logits, axis=1, keepdims=True)
        i1 = jnp.min(jnp.where(logits == m1, lane8, _E), axis=1,
                     keepdims=True)
        oh1 = lane8 == i1
        masked = jnp.where(oh1, -jnp.inf, logits)
        m2 = jnp.max(masked, axis=1, keepdims=True)
        i2 = jnp.min(jnp.where(masked == m2, lane8, _E), axis=1,
                     keepdims=True)
        oh2 = lane8 == i2
        sel = oh1 | oh2

        p = jnp.exp(logits - m1)
        p = p / jnp.sum(p, axis=1, keepdims=True)
        pm = jnp.where(sel, p, 0.0)
        gates_scr[...] = pm / (jnp.sum(pm, axis=1, keepdims=True) + 1e-9)
        ee_scr[...] = jnp.where(lane8 == 0, i1,
                                jnp.where(lane8 == 1, i2, 0))
        ee_dma = pltpu.make_async_copy(ee_scr, ee_smem, sem_ee.at[0])
        ee_dma.start()

        n = len(wi_srcs)
        for k in range(n):
            if k + 1 < n:
                wi_cp(k + 1).start()
                wo_cp(k + 1).start()
            wi_cp(k).wait()
            _, _, h_, de = wi_srcs[k]
            wi_bf[de, pl.ds(h_ * _HW, _HW), :] = stg_i[k % 2].astype(
                jnp.bfloat16)
            wo_cp(k).wait()
            _, _, h2_, de2 = wo_srcs[k]
            wo_bf[de2, pl.ds(h2_ * _HO, _HO), :] = stg_o[k % 2].astype(
                jnp.bfloat16)
        ee_dma.wait()

    x = x_ref[0]
    xb = x.astype(jnp.bfloat16)

    def mlp(e):
        h = jnp.maximum(
            jnp.dot(xb, wi_bf[e], preferred_element_type=jnp.float32)
            + bi_ref[e], 0.0)
        o = (jnp.dot(h.astype(jnp.bfloat16), wo_bf[e],
                     preferred_element_type=jnp.float32)
             + bo_ref[e] + x)
        mu = jnp.mean(o, axis=1, keepdims=True)
        var = jnp.mean((o - mu) ** 2, axis=1, keepdims=True)
        return (o - mu) / jnp.sqrt(var + 1e-5) * lg_ref[e] + lb_ref[e]

    e0 = ee_smem[s, 0]
    e1 = ee_smem[s, 1]
    lane8 = jax.lax.broadcasted_iota(jnp.int32, (1, _E), 1)
    grow = gates_scr[pl.ds(s, 1), :]
    w0 = jnp.sum(jnp.where(lane8 == e0, grow, 0.0))
    w1 = jnp.sum(jnp.where(lane8 == e1, grow, 0.0))
    tot = mlp(e0) * w0 + mlp(e1) * w1
    tot = tot.astype(jnp.bfloat16).astype(jnp.float32)
    out_ref[0] = mlp(_E) + tot


def kernel(cycle_curve_data, cycle_numbers, DKP_embeddings, gate_W1, gate_b1,
           gate_W2, gate_b2, gate_W3, gate_b3, exp_Wi, exp_bi, exp_Wo, exp_bo,
           exp_g, exp_b, gen_Wi, gen_bi, gen_Wo, gen_bo, gen_g, gen_b):
    bi_all = jnp.concatenate([exp_bi, gen_bi[None]], axis=0)[:, None, :]
    bo_all = jnp.concatenate([exp_bo, gen_bo[None]], axis=0)[:, None, :]
    lg_all = jnp.concatenate([exp_g, gen_g[None]], axis=0)[:, None, :]
    lb_all = jnp.concatenate([exp_b, gen_b[None]], axis=0)[:, None, :]

    _c = lambda idx: pl.BlockSpec(memory_space=pl.ANY)
    final = pl.pallas_call(
        _moe_kernel,
        grid=(_B,),
        in_specs=[
            pl.BlockSpec((1, _L, _DM), lambda s: (s, 0, 0)),
            pl.BlockSpec((_B, _DL), lambda s: (0, 0)),
            pl.BlockSpec((_B, 1), lambda s: (0, 0)),
            pl.BlockSpec(memory_space=pl.ANY),
            pl.BlockSpec((1, _DF), lambda s: (0, 0)),
            pl.BlockSpec((1, _DF), lambda s: (0, 0)),
            pl.BlockSpec((1, _DF), lambda s: (0, 0)),
            pl.BlockSpec((_DF, _E), lambda s: (0, 0)),
            pl.BlockSpec((1, _E), lambda s: (0, 0)),
            pl.BlockSpec(memory_space=pl.ANY),
            pl.BlockSpec(memory_space=pl.ANY),
            pl.BlockSpec(memory_space=pl.ANY),
            pl.BlockSpec(memory_space=pl.ANY),
            pl.BlockSpec((_E + 1, 1, _DF), lambda s: (0, 0, 0)),
            pl.BlockSpec((_E + 1, 1, _DM), lambda s: (0, 0, 0)),
            pl.BlockSpec((_E + 1, 1, _DM), lambda s: (0, 0, 0)),
            pl.BlockSpec((_E + 1, 1, _DM), lambda s: (0, 0, 0)),
        ],
        out_specs=pl.BlockSpec((1, _L, _DM), lambda s: (s, 0, 0)),
        scratch_shapes=[
            pltpu.VMEM((_E + 1, _DM, _DF), jnp.bfloat16),
            pltpu.VMEM((_E + 1, _DF, _DM), jnp.bfloat16),
            pltpu.VMEM((2, _HW, _DF), jnp.float32),
            pltpu.VMEM((2, _HO, _DM), jnp.float32),
            pltpu.VMEM((2, _C1, _DF), jnp.float32),
            pltpu.VMEM((_B, _E), jnp.float32),
            pltpu.VMEM((_B, _E), jnp.int32),
            pltpu.SMEM((_B, _E), jnp.int32),
            pltpu.SemaphoreType.DMA((2,)),
            pltpu.SemaphoreType.DMA((2,)),
            pltpu.SemaphoreType.DMA((2,)),
            pltpu.SemaphoreType.DMA((1,)),
        ],
        out_shape=jax.ShapeDtypeStruct((_B, _L, _DM), jnp.float32),
    )(cycle_curve_data, DKP_embeddings, cycle_numbers, gate_W1,
      gate_b1.reshape(1, _DF), gate_W2, gate_b2.reshape(1, _DF), gate_W3,
      gate_b3.reshape(1, _E), exp_Wi, exp_Wo,
      gen_Wi.reshape(1, _DM, _DF), gen_Wo.reshape(1, _DF, _DM),
      bi_all, bo_all, lg_all, lb_all)

    return (final, jnp.float32(0.0))


# speedup vs baseline: 3.1919x; 1.0783x over previous
import jax
import jax.numpy as jnp
from jax.experimental import pallas as pl
from jax.experimental.pallas import tpu as pltpu

_B = 64
_L = 128
_DM = 768
_DF = 1536
_DL = 2048
_E = 8

_HW = _DM // 4
_HO = _DF // 2
_C1 = 128
_SPB = 2


def _moe_kernel(x_ref, dkp_ref, cn_ref, w1_ref, b1_ref, w2_ref, b2_ref,
                w3_ref, b3_ref, ewi_ref, ewo_ref, gwi_ref, gwo_ref,
                bi_ref, bo_ref, lg_ref, lb_ref, out_ref,
                wi_bf, wo_bf, stg_i, stg_o, stg_w1, gates_scr, ee_scr,
                ee_smem, sem_i, sem_o, sem_w1, sem_ee):
    s = pl.program_id(0)

    @pl.when(s == 0)
    def _prologue():
        wi_srcs = ([(ewi_ref, e, h, e) for e in range(_E) for h in range(4)]
                   + [(gwi_ref, 0, h, _E) for h in range(4)])
        wo_srcs = ([(ewo_ref, e, h, e) for e in range(_E) for h in range(2)]
                   + [(gwo_ref, 0, h, _E) for h in range(2)])

        def wi_cp(k):
            src, se, h, _ = wi_srcs[k]
            return pltpu.make_async_copy(
                src.at[se, pl.ds(h * _HW, _HW), :], stg_i.at[k % 2],
                sem_i.at[k % 2])

        def wo_cp(k):
            src, se, h, _ = wo_srcs[k]
            return pltpu.make_async_copy(
                src.at[se, pl.ds(h * _HO, _HO), :], stg_o.at[k % 2],
                sem_o.at[k % 2])

        def w1_cp(c):
            return pltpu.make_async_copy(
                w1_ref.at[pl.ds(c * _C1, _C1), :], stg_w1.at[c % 2],
                sem_w1.at[c % 2])

        wi_cp(0).start()
        wo_cp(0).start()
        w1_cp(0).start()
        w1_cp(1).start()

        dkp = dkp_ref[...]
        nc = _DL // _C1
        h_acc = jnp.zeros((_B, _DF), jnp.float32)
        for c in range(nc):
            w1_cp(c).wait()
            chunk = stg_w1[c % 2]
            h_acc = h_acc + jnp.dot(
                dkp[:, c * _C1:(c + 1) * _C1], chunk,
                preferred_element_type=jnp.float32)
            if c + 2 < nc:
                w1_cp(c + 2).start()
        h1 = jnp.maximum(h_acc + b1_ref[...], 0.0)
        h2 = jnp.maximum(cn_ref[...] * w2_ref[...] + b2_ref[...], 0.0)
        h = h1 + h2
        logits = (jnp.dot(h, w3_ref[...], preferred_element_type=jnp.float32)
                  + b3_ref[...])

        lane8 = jax.lax.broadcasted_iota(jnp.int32, (_B, _E), 1)
        m1 = jnp.max(logits, axis=1, keepdims=True)
        i1 = jnp.min(jnp.where(logits == m1, lane8, _E), axis=1,
                     keepdims=True)
        oh1 = lane8 == i1
        masked = jnp.where(oh1, -jnp.inf, logits)
        m2 = jnp.max(masked, axis=1, keepdims=True)
        i2 = jnp.min(jnp.where(masked == m2, lane8, _E), axis=1,
                     keepdims=True)
        oh2 = lane8 == i2
        sel = oh1 | oh2

        p = jnp.exp(logits - m1)
        p = p / jnp.sum(p, axis=1, keepdims=True)
        pm = jnp.where(sel, p, 0.0)
        gates_scr[...] = pm / (jnp.sum(pm, axis=1, keepdims=True) + 1e-9)
        ee_scr[...] = jnp.where(lane8 == 0, i1,
                                jnp.where(lane8 == 1, i2, 0))
        ee_dma = pltpu.make_async_copy(ee_scr, ee_smem, sem_ee.at[0])
        ee_dma.start()

        ni = len(wi_srcs)
        no = len(wo_srcs)
        for k in range(ni):
            if k + 1 < ni:
                wi_cp(k + 1).start()
            if k % 2 == 0 and k // 2 + 1 < no:
                wo_cp(k // 2 + 1).start()
            wi_cp(k).wait()
            _, _, h_, de = wi_srcs[k]
            wi_bf[de, pl.ds(h_ * _HW, _HW), :] = stg_i[k % 2].astype(
                jnp.bfloat16)
            if k % 2 == 1:
                ko = k // 2
                wo_cp(ko).wait()
                _, _, h2_, de2 = wo_srcs[ko]
                wo_bf[de2, pl.ds(h2_ * _HO, _HO), :] = stg_o[ko % 2].astype(
                    jnp.bfloat16)
        ee_dma.wait()

    def mlp_of(xv, xbv, e):
        h = jnp.maximum(
            jnp.dot(xbv, wi_bf[e], preferred_element_type=jnp.float32)
            + bi_ref[e], 0.0)
        o = (jnp.dot(h.astype(jnp.bfloat16), wo_bf[e],
                     preferred_element_type=jnp.float32)
             + bo_ref[e] + xv)
        mu = jnp.mean(o, axis=1, keepdims=True)
        var = jnp.mean((o - mu) ** 2, axis=1, keepdims=True)
        return (o - mu) / jnp.sqrt(var + 1e-5) * lg_ref[e] + lb_ref[e]

    xf = x_ref[...].reshape(_SPB * _L, _DM)
    xbf = xf.astype(jnp.bfloat16)
    gen = mlp_of(xf, xbf, _E)

    lane8 = jax.lax.broadcasted_iota(jnp.int32, (1, _E), 1)
    for i in range(_SPB):
        xi = xf[i * _L:(i + 1) * _L]
        xbi = xbf[i * _L:(i + 1) * _L]
        e0 = ee_smem[s * _SPB + i, 0]
        e1 = ee_smem[s * _SPB + i, 1]
        grow = gates_scr[pl.ds(s * _SPB + i, 1), :]
        w0 = jnp.sum(jnp.where(lane8 == e0, grow, 0.0))
        w1 = jnp.sum(jnp.where(lane8 == e1, grow, 0.0))
        tot = mlp_of(xi, xbi, e0) * w0 + mlp_of(xi, xbi, e1) * w1
        tot = tot.astype(jnp.bfloat16).astype(jnp.float32)
        out_ref[i] = gen[i * _L:(i + 1) * _L] + tot


def kernel(cycle_curve_data, cycle_numbers, DKP_embeddings, gate_W1, gate_b1,
           gate_W2, gate_b2, gate_W3, gate_b3, exp_Wi, exp_bi, exp_Wo, exp_bo,
           exp_g, exp_b, gen_Wi, gen_bi, gen_Wo, gen_bo, gen_g, gen_b):
    bi_all = jnp.concatenate([exp_bi, gen_bi[None]], axis=0)[:, None, :]
    bo_all = jnp.concatenate([exp_bo, gen_bo[None]], axis=0)[:, None, :]
    lg_all = jnp.concatenate([exp_g, gen_g[None]], axis=0)[:, None, :]
    lb_all = jnp.concatenate([exp_b, gen_b[None]], axis=0)[:, None, :]

    _c = lambda idx: pl.BlockSpec(memory_space=pl.ANY)
    final = pl.pallas_call(
        _moe_kernel,
        grid=(_B // _SPB,),
        in_specs=[
            pl.BlockSpec((_SPB, _L, _DM), lambda s: (s, 0, 0)),
            pl.BlockSpec((_B, _DL), lambda s: (0, 0)),
            pl.BlockSpec((_B, 1), lambda s: (0, 0)),
            pl.BlockSpec(memory_space=pl.ANY),
            pl.BlockSpec((1, _DF), lambda s: (0, 0)),
            pl.BlockSpec((1, _DF), lambda s: (0, 0)),
            pl.BlockSpec((1, _DF), lambda s: (0, 0)),
            pl.BlockSpec((_DF, _E), lambda s: (0, 0)),
            pl.BlockSpec((1, _E), lambda s: (0, 0)),
            pl.BlockSpec(memory_space=pl.ANY),
            pl.BlockSpec(memory_space=pl.ANY),
            pl.BlockSpec(memory_space=pl.ANY),
            pl.BlockSpec(memory_space=pl.ANY),
            pl.BlockSpec((_E + 1, 1, _DF), lambda s: (0, 0, 0)),
            pl.BlockSpec((_E + 1, 1, _DM), lambda s: (0, 0, 0)),
            pl.BlockSpec((_E + 1, 1, _DM), lambda s: (0, 0, 0)),
            pl.BlockSpec((_E + 1, 1, _DM), lambda s: (0, 0, 0)),
        ],
        out_specs=pl.BlockSpec((_SPB, _L, _DM), lambda s: (s, 0, 0)),
        scratch_shapes=[
            pltpu.VMEM((_E + 1, _DM, _DF), jnp.bfloat16),
            pltpu.VMEM((_E + 1, _DF, _DM), jnp.bfloat16),
            pltpu.VMEM((2, _HW, _DF), jnp.float32),
            pltpu.VMEM((2, _HO, _DM), jnp.float32),
            pltpu.VMEM((2, _C1, _DF), jnp.float32),
            pltpu.VMEM((_B, _E), jnp.float32),
            pltpu.VMEM((_B, _E), jnp.int32),
            pltpu.SMEM((_B, _E), jnp.int32),
            pltpu.SemaphoreType.DMA((2,)),
            pltpu.SemaphoreType.DMA((2,)),
            pltpu.SemaphoreType.DMA((2,)),
            pltpu.SemaphoreType.DMA((1,)),
        ],
        out_shape=jax.ShapeDtypeStruct((_B, _L, _DM), jnp.float32),
    )(cycle_curve_data, DKP_embeddings, cycle_numbers, gate_W1,
      gate_b1.reshape(1, _DF), gate_W2, gate_b2.reshape(1, _DF), gate_W3,
      gate_b3.reshape(1, _E), exp_Wi, exp_Wo,
      gen_Wi.reshape(1, _DM, _DF), gen_Wo.reshape(1, _DF, _DM),
      bi_all, bo_all, lg_all, lb_all)

    return (final, jnp.float32(0.0))
